# baseline (speedup 1.0000x reference)
import functools

import jax
import jax.numpy as jnp
import numpy as np
from jax.experimental import pallas as pl
from jax.experimental.pallas import tpu as pltpu

B, S, D = 1, 2048, 2048
H, KVH, HD = 16, 4, 128
E, K, F = 8, 2, 768
EPS = 1e-06
THETA = 10000.0
HALF = HD // 2

TS = 256
TNQ = KVH * HD
NQKV = (H + 2 * KVH) * HD
N_QB = (H * HD) // TNQ
TQ = 256
TK = 256
SCALE = 1.0 / float(np.sqrt(HD))


def _rms(x):
    return jax.lax.rsqrt(jnp.mean(x * x, axis=-1, keepdims=True) + EPS)


def _qkv_body(x_ref, ln1_ref, w_ref, qn_ref, kn_ref, o_ref):
    i = pl.program_id(0)
    j = pl.program_id(1)
    x = x_ref[...]
    h = x * _rms(x) * ln1_ref[...]
    y = jnp.dot(h, w_ref[...], preferred_element_type=jnp.float32)
    nh = TNQ // HD
    y3 = y.reshape(TS, nh, HD)
    is_k = j == N_QB
    is_v = j == N_QB + 1
    w_head = jnp.where(is_k, kn_ref[...], qn_ref[...])
    yn = y3 * _rms(y3) * w_head[None]
    pos = (i * TS + jax.lax.broadcasted_iota(jnp.int32, (TS, HALF), 0)
           ).astype(jnp.float32)
    inv = jnp.exp(jax.lax.broadcasted_iota(jnp.int32, (TS, HALF), 1)
                  .astype(jnp.float32) * (-np.log(THETA) / HALF))
    f = pos * inv
    cos = jnp.cos(f)[:, None, :]
    sin = jnp.sin(f)[:, None, :]
    x1 = yn[..., :HALF]
    x2 = yn[..., HALF:]
    rot = jnp.concatenate([x1 * cos - x2 * sin, x2 * cos + x1 * sin], axis=-1)
    o_ref[...] = jnp.where(is_v, y3, rot).reshape(TS, TNQ)


def _qkv(x, ln1_w, w_all, qn, kn):
    return pl.pallas_call(
        _qkv_body,
        grid=(S // TS, NQKV // TNQ),
        in_specs=[
            pl.BlockSpec((TS, D), lambda i, j: (i, 0)),
            pl.BlockSpec((1, D), lambda i, j: (0, 0)),
            pl.BlockSpec((D, TNQ), lambda i, j: (0, j)),
            pl.BlockSpec((1, HD), lambda i, j: (0, 0)),
            pl.BlockSpec((1, HD), lambda i, j: (0, 0)),
        ],
        out_specs=pl.BlockSpec((TS, TNQ), lambda i, j: (i, j)),
        out_shape=jax.ShapeDtypeStruct((S, NQKV), jnp.float32),
    )(x, ln1_w.reshape(1, D), w_all, qn.reshape(1, HD), kn.reshape(1, HD))


def _attn_body(q_ref, k_ref, v_ref, o_ref):
    i = pl.program_id(1)
    q = q_ref[...] * SCALE
    riota = jax.lax.broadcasted_iota(jnp.int32, (TQ, TK), 0)
    ciota = jax.lax.broadcasted_iota(jnp.int32, (TQ, TK), 1)

    def step(kt, carry):
        m, l, acc = carry
        kb = k_ref[pl.ds(kt * TK, TK), :]
        vb = v_ref[pl.ds(kt * TK, TK), :]
        s = jax.lax.dot_general(q, kb, (((1,), (1,)), ((), ())),
                                preferred_element_type=jnp.float32)
        keep = (kt < i) | (riota >= ciota)
        s = jnp.where(keep, s, -1e30)
        m_new = jnp.maximum(m, jnp.max(s, axis=1, keepdims=True))
        alpha = jnp.exp(m - m_new)
        p = jnp.exp(s - m_new)
        l_new = l * alpha + jnp.sum(p, axis=1, keepdims=True)
        acc_new = acc * alpha + jnp.dot(p, vb, preferred_element_type=jnp.float32)
        return m_new, l_new, acc_new

    m0 = jnp.full((TQ, 1), -1e30, jnp.float32)
    l0 = jnp.zeros((TQ, 1), jnp.float32)
    a0 = jnp.zeros((TQ, HD), jnp.float32)
    m, l, acc = jax.lax.fori_loop(0, i + 1, step, (m0, l0, a0))
    o_ref[...] = acc / l


def _attn(qkv):
    rep = H // KVH
    return pl.pallas_call(
        _attn_body,
        grid=(H, S // TQ),
        in_specs=[
            pl.BlockSpec((TQ, HD), lambda h, i: (i, h)),
            pl.BlockSpec((S, HD), lambda h, i: (0, H + h // rep)),
            pl.BlockSpec((S, HD), lambda h, i: (0, H + KVH + h // rep)),
        ],
        out_specs=pl.BlockSpec((TQ, HD), lambda h, i: (i, h)),
        out_shape=jax.ShapeDtypeStruct((S, H * HD), jnp.float32),
    )(qkv, qkv, qkv)


TNO = 512


def _wo_body(o_ref, w_ref, r_ref, y_ref):
    y_ref[...] = r_ref[...] + jnp.dot(o_ref[...], w_ref[...],
                                      preferred_element_type=jnp.float32)


def _wo(o, wo, resid):
    return pl.pallas_call(
        _wo_body,
        grid=(S // TS, D // TNO),
        in_specs=[
            pl.BlockSpec((TS, H * HD), lambda i, j: (i, 0)),
            pl.BlockSpec((H * HD, TNO), lambda i, j: (0, j)),
            pl.BlockSpec((TS, TNO), lambda i, j: (i, j)),
        ],
        out_specs=pl.BlockSpec((TS, TNO), lambda i, j: (i, j)),
        out_shape=jax.ShapeDtypeStruct((S, D), jnp.float32),
    )(o, wo, resid)


def _ln2_body(x_ref, w_ref, rw_ref, h_ref, lg_ref):
    x = x_ref[...]
    hh = x * _rms(x) * w_ref[...]
    h_ref[...] = hh
    lg_ref[...] = jnp.dot(hh, rw_ref[...], preferred_element_type=jnp.float32)


def _ln2(x, ln2_w, router_W):
    return pl.pallas_call(
        _ln2_body,
        grid=(S // TS,),
        in_specs=[
            pl.BlockSpec((TS, D), lambda i: (i, 0)),
            pl.BlockSpec((1, D), lambda i: (0, 0)),
            pl.BlockSpec((D, E), lambda i: (0, 0)),
        ],
        out_specs=[
            pl.BlockSpec((TS, D), lambda i: (i, 0)),
            pl.BlockSpec((TS, E), lambda i: (i, 0)),
        ],
        out_shape=[
            jax.ShapeDtypeStruct((S, D), jnp.float32),
            jax.ShapeDtypeStruct((S, E), jnp.float32),
        ],
    )(x, ln2_w.reshape(1, D), router_W)


def _route_body(lg_ref, coef_ref):
    lg = lg_ref[...]
    m = jnp.max(lg, axis=1, keepdims=True)
    p = jnp.exp(lg - m)
    p = p / jnp.sum(p, axis=1, keepdims=True)
    iota = jax.lax.broadcasted_iota(jnp.int32, (S, E), 1)
    m1 = jnp.max(p, axis=1, keepdims=True)
    i1 = jnp.min(jnp.where(p == m1, iota, E), axis=1, keepdims=True)
    oh1 = iota == i1
    p2 = jnp.where(oh1, -1.0, p)
    m2 = jnp.max(p2, axis=1, keepdims=True)
    i2 = jnp.min(jnp.where(p2 == m2, iota, E), axis=1, keepdims=True)
    oh2 = iota == i2
    denom = m1 + m2
    coef_ref[...] = (jnp.where(oh1, m1, 0.0) + jnp.where(oh2, m2, 0.0)) / denom


def _route(logits):
    return pl.pallas_call(
        _route_body,
        out_shape=jax.ShapeDtypeStruct((S, E), jnp.float32),
    )(logits)


def _moe_body(h_ref, c_ref, x_ref, wg_ref, wu_ref, wd_ref, o_ref):
    e = pl.program_id(1)
    h = h_ref[...]
    g = jnp.dot(h, wg_ref[0], preferred_element_type=jnp.float32)
    u = jnp.dot(h, wu_ref[0], preferred_element_type=jnp.float32)
    y = jnp.dot(g * jax.nn.sigmoid(g) * u, wd_ref[0],
                preferred_element_type=jnp.float32)
    iota = jax.lax.broadcasted_iota(jnp.int32, (TS, E), 1)
    c = jnp.sum(jnp.where(iota == e, c_ref[...], 0.0), axis=1, keepdims=True)
    contrib = c * y

    @pl.when(e == 0)
    def _():
        o_ref[...] = x_ref[...] + contrib

    @pl.when(e != 0)
    def _():
        o_ref[...] += contrib


def _moe(h2, coef, x2, wg, wu, wd):
    return pl.pallas_call(
        _moe_body,
        grid=(S // TS, E),
        in_specs=[
            pl.BlockSpec((TS, D), lambda i, e: (i, 0)),
            pl.BlockSpec((TS, E), lambda i, e: (i, 0)),
            pl.BlockSpec((TS, D), lambda i, e: (i, 0)),
            pl.BlockSpec((1, D, F), lambda i, e: (e, 0, 0)),
            pl.BlockSpec((1, D, F), lambda i, e: (e, 0, 0)),
            pl.BlockSpec((1, F, D), lambda i, e: (e, 0, 0)),
        ],
        out_specs=pl.BlockSpec((TS, D), lambda i, e: (i, 0)),
        out_shape=jax.ShapeDtypeStruct((S, D), jnp.float32),
    )(h2, coef, x2, wg, wu, wd)


def kernel(hidden_states, ln1_w, Wq, Wk, Wv, q_norm_w, k_norm_w, Wo, ln2_w,
           router_W, W_gate, W_up, W_down):
    x = hidden_states.reshape(S, D)
    w_all = jnp.concatenate([Wq, Wk, Wv], axis=1)
    qkv = _qkv(x, ln1_w, w_all, q_norm_w, k_norm_w)
    o = _attn(qkv)
    x2 = _wo(o, Wo, x)
    h2, logits = _ln2(x2, ln2_w, router_W)
    coef = _route(logits)
    out = _moe(h2, coef, x2, W_gate, W_up, W_down)
    return out.reshape(B, S, D)
